# Initial kernel scaffold; baseline (speedup 1.0000x reference)
import jax
import jax.numpy as jnp
from jax.experimental import pallas as pl


def kernel(labels, embedding_table):
    raise NotImplementedError("write your pallas kernel here")


# speedup vs baseline: 2.2196x; 2.2196x over previous
import functools

import jax
import jax.numpy as jnp
from jax import lax
from jax.experimental import pallas as pl
from jax.experimental.pallas import tpu as pltpu
from jax.experimental.pallas import tpu_sc as plsc

NUM_CLASSES = 1000
HIDDEN = 128
BATCH = 16384

_NC = 2
_NS = 16
_NW = _NC * _NS
_BPW = BATCH // _NW
_CHUNK = 128
_NCHUNK = _BPW // _CHUNK


def _embed_body(labels_hbm, table_hbm, out_hbm, idx_v, rows_v, gsem, ssem):
    wid = lax.axis_index("s") * _NC + lax.axis_index("c")
    base = wid * _BPW

    for j in range(_NCHUNK):
        pltpu.sync_copy(labels_hbm.at[pl.ds(base + j * _CHUNK, _CHUNK)],
                        idx_v.at[j])

    gathers = []
    for j in range(_NCHUNK):
        gathers.append(
            pltpu.async_copy(table_hbm.at[idx_v.at[j]],
                             rows_v.at[pl.ds(j * _CHUNK, _CHUNK), :],
                             gsem))
    scatters = []
    for j in range(_NCHUNK):
        gathers[j].wait()
        scatters.append(
            pltpu.async_copy(rows_v.at[pl.ds(j * _CHUNK, _CHUNK), :],
                             out_hbm.at[pl.ds(base + j * _CHUNK, _CHUNK), :],
                             ssem))
    for s in scatters:
        s.wait()


@jax.jit
def _embed(labels, table):
    mesh = plsc.VectorSubcoreMesh(core_axis_name="c", subcore_axis_name="s")
    return pl.kernel(
        _embed_body,
        out_type=jax.ShapeDtypeStruct((BATCH, HIDDEN), jnp.float32),
        mesh=mesh,
        scratch_types=[
            pltpu.VMEM((_NCHUNK, _CHUNK), jnp.int32),
            pltpu.VMEM((_BPW, HIDDEN), jnp.float32),
            pltpu.SemaphoreType.DMA,
            pltpu.SemaphoreType.DMA,
        ],
    )(labels, table)


def kernel(labels, embedding_table):
    return _embed(labels, embedding_table)


# speedup vs baseline: 2.3138x; 1.0424x over previous
import functools

import jax
import jax.numpy as jnp
from jax import lax
from jax.experimental import pallas as pl
from jax.experimental.pallas import tpu as pltpu
from jax.experimental.pallas import tpu_sc as plsc

NUM_CLASSES = 1000
HIDDEN = 128
BATCH = 16384

_NC = 2
_NS = 16
_NW = _NC * _NS
_BPW = BATCH // _NW
_CHUNK = 128
_NCHUNK = _BPW // _CHUNK


def _embed_body(labels_hbm, table_hbm, out_hbm, idx_v, rows_v, gsem, ssem):
    wid = lax.axis_index("s") * _NC + lax.axis_index("c")
    base = wid * _BPW

    pltpu.sync_copy(labels_hbm.at[pl.ds(wid * _NCHUNK, _NCHUNK), :], idx_v)

    gathers = []
    for j in range(_NCHUNK):
        gathers.append(
            pltpu.async_copy(table_hbm.at[idx_v.at[j]],
                             rows_v.at[pl.ds(j * _CHUNK, _CHUNK), :],
                             gsem))
    scatters = []
    for j in range(_NCHUNK):
        gathers[j].wait()
        scatters.append(
            pltpu.async_copy(rows_v.at[pl.ds(j * _CHUNK, _CHUNK), :],
                             out_hbm.at[pl.ds(base + j * _CHUNK, _CHUNK), :],
                             ssem))
    for s in scatters:
        s.wait()


@jax.jit
def _embed(labels, table):
    mesh = plsc.VectorSubcoreMesh(core_axis_name="c", subcore_axis_name="s")
    return pl.kernel(
        _embed_body,
        out_type=jax.ShapeDtypeStruct((BATCH, HIDDEN), jnp.float32),
        mesh=mesh,
        scratch_types=[
            pltpu.VMEM((_NCHUNK, _CHUNK), jnp.int32),
            pltpu.VMEM((_BPW, HIDDEN), jnp.float32),
            pltpu.SemaphoreType.DMA,
            pltpu.SemaphoreType.DMA,
        ],
    )(labels.reshape(BATCH // _CHUNK, _CHUNK), table)


def kernel(labels, embedding_table):
    return _embed(labels, embedding_table)
